# Initial kernel scaffold; baseline (speedup 1.0000x reference)
import jax
import jax.numpy as jnp
from jax.experimental import pallas as pl


def kernel(student_id, engagement_id, student_features, engagement_features, student_table, engagement_table, s_W1, s_b1, s_W2, s_b2, s_W3, s_b3, e_W1, e_b1, e_W2, e_b2, e_W3, e_b3, rank_W, rank_b, like_W, like_b, risk_W, risk_b):
    raise NotImplementedError("write your pallas kernel here")


# speedup vs baseline: 1.3601x; 1.3601x over previous
import functools

import jax
import jax.numpy as jnp
from jax import lax
from jax.experimental import pallas as pl
from jax.experimental.pallas import tpu as pltpu
from jax.experimental.pallas import tpu_sc as plsc

B = 16384
VOCAB = 1001
D = 64
F = 10

_NC = 2
_NS = 16
_L = 16
_NW = _NC * _NS
_BPW = B // _NW

_DP = 128
_CH = 128
_NCH = _BPW // _CH

_sc_mesh = plsc.VectorSubcoreMesh(core_axis_name="c", subcore_axis_name="s")


@functools.partial(
    pl.kernel,
    mesh=_sc_mesh,
    out_type=[
        jax.ShapeDtypeStruct((B, _DP), jnp.float32),
        jax.ShapeDtypeStruct((B, _DP), jnp.float32),
    ],
    scratch_types=[
        pltpu.VMEM((_BPW,), jnp.int32),
        pltpu.VMEM((_BPW,), jnp.int32),
        pltpu.VMEM((2, _CH, _DP), jnp.float32),
        pltpu.SemaphoreType.DMA,
        pltpu.SemaphoreType.DMA,
    ],
)
def _sc_gather(s_table, e_table, s_id, e_id, s_out, e_out,
               s_idx_v, e_idx_v, rows_v, sem0, sem1):
    wid = lax.axis_index("s") * _NC + lax.axis_index("c")
    base = wid * _BPW

    pltpu.sync_copy(s_id.at[pl.ds(base, _BPW)], s_idx_v)
    pltpu.sync_copy(e_id.at[pl.ds(base, _BPW)], e_idx_v)

    def _mod_body(i, carry):
        sl = pl.ds(i * _L, _L)
        s_idx_v[sl] = lax.rem(s_idx_v[sl], VOCAB)
        e_idx_v[sl] = lax.rem(e_idx_v[sl], VOCAB)
        return carry

    lax.fori_loop(0, _BPW // _L, _mod_body, 0)

    sems = (sem0, sem1)
    def _gather(c, buf):
        if c < _NCH:
            table, idx = s_table, s_idx_v.at[pl.ds(c * _CH, _CH)]
        else:
            table, idx = e_table, e_idx_v.at[pl.ds((c - _NCH) * _CH, _CH)]
        return pltpu.async_copy(table.at[idx], rows_v.at[buf], sems[buf])

    cps = [_gather(0, 0), _gather(1, 1)]
    for c in range(2 * _NCH):
        buf = c % 2
        cps[buf].wait()
        if c + 2 < 2 * _NCH:
            nxt = c + 2
        else:
            nxt = None
        if c < _NCH:
            out, row0 = s_out, base + c * _CH
        else:
            out, row0 = e_out, base + (c - _NCH) * _CH
        pltpu.sync_copy(rows_v.at[buf], out.at[pl.ds(row0, _CH)])
        if nxt is not None:
            cps[buf] = _gather(nxt, buf)


def _l2norm(x):
    sq = jnp.sum(x * x, axis=-1, keepdims=True)
    return x * lax.rsqrt(jnp.maximum(sq, 1e-12))


def _tower(x, W1, b1, W2, b2, W3, b3):
    h = jnp.maximum(jnp.dot(x, W1, preferred_element_type=jnp.float32) + b1, 0.0)
    h = jnp.maximum(jnp.dot(h, W2, preferred_element_type=jnp.float32) + b2, 0.0)
    return jnp.maximum(jnp.dot(h, W3, preferred_element_type=jnp.float32) + b3, 0.0)


def _tc_body(sf_ref, ef_ref, sb_ref, eb_ref,
             sW1, sb1, sW2, sb2, sW3, sb3,
             eW1, eb1, eW2, eb2, eW3, eb3,
             hW, hb, out_ref):
    s_feat = _tower(sf_ref[...], sW1[...], sb1[...], sW2[...], sb2[...],
                    sW3[...], sb3[...])
    e_feat = _tower(ef_ref[...], eW1[...], eb1[...], eW2[...], eb2[...],
                    eW3[...], eb3[...])
    s_emb = _l2norm(sb_ref[:, :D] + s_feat)
    e_emb = _l2norm(eb_ref[:, :D] + e_feat)
    combined = s_emb * e_emb
    z = jnp.dot(combined, hW[...], preferred_element_type=jnp.float32) + hb[...]
    out_ref[...] = 1.0 / (1.0 + jnp.exp(-z))


_R = 2048


def _tc_dense(sf, ef, sb, eb, weights):
    (sW1, sb1, sW2, sb2, sW3, sb3,
     eW1, eb1, eW2, eb2, eW3, eb3, hW, hb) = weights

    def _full(a):
        return pl.BlockSpec(a.shape, lambda i: (0,) * a.ndim)

    row_spec = lambda a: pl.BlockSpec((_R, a.shape[1]), lambda i: (i, 0))
    base_spec = pl.BlockSpec((_R, _DP), lambda i: (i, 0))
    in_specs = [row_spec(sf), row_spec(ef), base_spec, base_spec] + [
        _full(w) for w in (sW1, sb1, sW2, sb2, sW3, sb3,
                           eW1, eb1, eW2, eb2, eW3, eb3, hW, hb)
    ]
    return pl.pallas_call(
        _tc_body,
        grid=(B // _R,),
        in_specs=in_specs,
        out_specs=pl.BlockSpec((_R, 3), lambda i: (i, 0)),
        out_shape=jax.ShapeDtypeStruct((B, 3), jnp.float32),
    )(sf, ef, sb, eb, sW1, sb1, sW2, sb2, sW3, sb3,
      eW1, eb1, eW2, eb2, eW3, eb3, hW, hb)


def kernel(student_id, engagement_id, student_features, engagement_features,
           student_table, engagement_table,
           s_W1, s_b1, s_W2, s_b2, s_W3, s_b3,
           e_W1, e_b1, e_W2, e_b2, e_W3, e_b3,
           rank_W, rank_b, like_W, like_b, risk_W, risk_b):
    s_table_p = jnp.pad(student_table, ((0, 0), (0, _DP - D)))
    e_table_p = jnp.pad(engagement_table, ((0, 0), (0, _DP - D)))
    s_base, e_base = _sc_gather(s_table_p, e_table_p,
                                student_id, engagement_id)
    head_W = jnp.concatenate([rank_W, like_W, risk_W], axis=1)
    head_b = jnp.concatenate([rank_b, like_b, risk_b]).reshape(1, 3)
    weights = (s_W1, s_b1.reshape(1, -1), s_W2, s_b2.reshape(1, -1),
               s_W3, s_b3.reshape(1, -1),
               e_W1, e_b1.reshape(1, -1), e_W2, e_b2.reshape(1, -1),
               e_W3, e_b3.reshape(1, -1), head_W, head_b)
    out = _tc_dense(student_features, engagement_features, s_base, e_base,
                    weights)
    return (out[:, 0:1], out[:, 1:2], out[:, 2:3])


# speedup vs baseline: 1.5205x; 1.1179x over previous
import functools

import jax
import jax.numpy as jnp
from jax import lax
from jax.experimental import pallas as pl
from jax.experimental.pallas import tpu as pltpu
from jax.experimental.pallas import tpu_sc as plsc

B = 16384
VOCAB = 1001
D = 64
F = 10

_NC = 2
_NS = 16
_L = 16
_NW = _NC * _NS
_BPW = B // _NW

_DP = 128
_CH = 128
_NCH = _BPW // _CH

_sc_mesh = plsc.VectorSubcoreMesh(core_axis_name="c", subcore_axis_name="s")


@functools.partial(
    pl.kernel,
    mesh=_sc_mesh,
    out_type=[
        jax.ShapeDtypeStruct((B, _DP), jnp.float32),
        jax.ShapeDtypeStruct((B, _DP), jnp.float32),
    ],
    scratch_types=[
        pltpu.VMEM((_BPW,), jnp.int32),
        pltpu.VMEM((_BPW,), jnp.int32),
        pltpu.VMEM((2, _CH, _DP), jnp.float32),
        pltpu.SemaphoreType.DMA,
        pltpu.SemaphoreType.DMA,
    ],
)
def _sc_gather(s_table, e_table, s_id, e_id, s_out, e_out,
               s_idx_v, e_idx_v, rows_v, sem0, sem1):
    wid = lax.axis_index("s") * _NC + lax.axis_index("c")
    base = wid * _BPW

    pltpu.sync_copy(s_id.at[pl.ds(base, _BPW)], s_idx_v)
    pltpu.sync_copy(e_id.at[pl.ds(base, _BPW)], e_idx_v)

    def _mod_body(i, carry):
        sl = pl.ds(i * _L, _L)
        s_idx_v[sl] = lax.rem(s_idx_v[sl], VOCAB)
        e_idx_v[sl] = lax.rem(e_idx_v[sl], VOCAB)
        return carry

    lax.fori_loop(0, _BPW // _L, _mod_body, 0)

    sems = (sem0, sem1)
    def _gather(c, buf):
        if c < _NCH:
            table, idx = s_table, s_idx_v.at[pl.ds(c * _CH, _CH)]
        else:
            table, idx = e_table, e_idx_v.at[pl.ds((c - _NCH) * _CH, _CH)]
        return pltpu.async_copy(table.at[idx], rows_v.at[buf], sems[buf])

    cps = [_gather(0, 0), _gather(1, 1)]
    for c in range(2 * _NCH):
        buf = c % 2
        cps[buf].wait()
        if c + 2 < 2 * _NCH:
            nxt = c + 2
        else:
            nxt = None
        if c < _NCH:
            out, row0 = s_out, base + c * _CH
        else:
            out, row0 = e_out, base + (c - _NCH) * _CH
        pltpu.sync_copy(rows_v.at[buf], out.at[pl.ds(row0, _CH)])
        if nxt is not None:
            cps[buf] = _gather(nxt, buf)


def _l2norm(x):
    sq = jnp.sum(x * x, axis=-1, keepdims=True)
    return x * lax.rsqrt(jnp.maximum(sq, 1e-12))


def _tower(x, W1, b1, W2, b2, W3, b3):
    h = jnp.maximum(jnp.dot(x, W1, preferred_element_type=jnp.float32) + b1, 0.0)
    h = jnp.maximum(jnp.dot(h, W2, preferred_element_type=jnp.float32) + b2, 0.0)
    return jnp.maximum(jnp.dot(h, W3, preferred_element_type=jnp.float32) + b3, 0.0)


def _tc_body(sf_ref, ef_ref, sb_ref, eb_ref,
             sW1, sb1, sW2, sb2, sW3, sb3,
             eW1, eb1, eW2, eb2, eW3, eb3,
             rW, rb, lW, lb, kW, kb,
             rank_ref, like_ref, risk_ref):
    s_feat = _tower(sf_ref[...], sW1[...], sb1[...], sW2[...], sb2[...],
                    sW3[...], sb3[...])
    e_feat = _tower(ef_ref[...], eW1[...], eb1[...], eW2[...], eb2[...],
                    eW3[...], eb3[...])
    s_emb = _l2norm(sb_ref[:, :D] + s_feat)
    e_emb = _l2norm(eb_ref[:, :D] + e_feat)
    combined = s_emb * e_emb

    def _head(W, b, out_ref):
        z = jnp.dot(combined, W[...], preferred_element_type=jnp.float32) + b[...]
        out_ref[...] = 1.0 / (1.0 + jnp.exp(-z))

    _head(rW, rb, rank_ref)
    _head(lW, lb, like_ref)
    _head(kW, kb, risk_ref)


_R = 2048


def _tc_dense(sf, ef, sb, eb, weights):
    def _full(a):
        return pl.BlockSpec(a.shape, lambda i: (0,) * a.ndim)

    row_spec = lambda a: pl.BlockSpec((_R, a.shape[1]), lambda i: (i, 0))
    base_spec = pl.BlockSpec((_R, _DP), lambda i: (i, 0))
    in_specs = [row_spec(sf), row_spec(ef), base_spec, base_spec] + [
        _full(w) for w in weights
    ]
    out_spec = pl.BlockSpec((_R, 1), lambda i: (i, 0))
    out_shape = jax.ShapeDtypeStruct((B, 1), jnp.float32)
    return pl.pallas_call(
        _tc_body,
        grid=(B // _R,),
        in_specs=in_specs,
        out_specs=[out_spec] * 3,
        out_shape=[out_shape] * 3,
    )(sf, ef, sb, eb, *weights)


def kernel(student_id, engagement_id, student_features, engagement_features,
           student_table, engagement_table,
           s_W1, s_b1, s_W2, s_b2, s_W3, s_b3,
           e_W1, e_b1, e_W2, e_b2, e_W3, e_b3,
           rank_W, rank_b, like_W, like_b, risk_W, risk_b):
    s_table_p = jnp.pad(student_table, ((0, 0), (0, _DP - D)))
    e_table_p = jnp.pad(engagement_table, ((0, 0), (0, _DP - D)))
    s_base, e_base = _sc_gather(s_table_p, e_table_p,
                                student_id, engagement_id)
    weights = (s_W1, s_b1, s_W2, s_b2, s_W3, s_b3,
               e_W1, e_b1, e_W2, e_b2, e_W3, e_b3,
               rank_W, rank_b, like_W, like_b, risk_W, risk_b)
    rank, like, risk = _tc_dense(student_features, engagement_features,
                                 s_base, e_base, weights)
    return (rank, like, risk)


# speedup vs baseline: 1.5877x; 1.0442x over previous
import functools

import jax
import jax.numpy as jnp
from jax import lax
from jax.experimental import pallas as pl
from jax.experimental.pallas import tpu as pltpu
from jax.experimental.pallas import tpu_sc as plsc

B = 16384
VOCAB = 1001
D = 64
F = 10

_NC = 2
_NS = 16
_L = 16
_NW = _NC * _NS
_BPW = B // _NW

_DP = 128
_CH = 128
_NCH = _BPW // _CH

_sc_mesh = plsc.VectorSubcoreMesh(core_axis_name="c", subcore_axis_name="s")


@functools.partial(
    pl.kernel,
    mesh=_sc_mesh,
    out_type=[
        jax.ShapeDtypeStruct((B, _DP), jnp.float32),
        jax.ShapeDtypeStruct((B, _DP), jnp.float32),
    ],
    scratch_types=[
        pltpu.VMEM((_BPW,), jnp.int32),
        pltpu.VMEM((_BPW,), jnp.int32),
        pltpu.VMEM((2, _CH, _DP), jnp.float32),
        pltpu.SemaphoreType.DMA,
        pltpu.SemaphoreType.DMA,
    ],
)
def _sc_gather(s_table, e_table, s_id, e_id, s_out, e_out,
               s_idx_v, e_idx_v, rows_v, sem0, sem1):
    wid = lax.axis_index("s") * _NC + lax.axis_index("c")
    base = wid * _BPW

    pltpu.sync_copy(s_id.at[pl.ds(base, _BPW)], s_idx_v)
    pltpu.sync_copy(e_id.at[pl.ds(base, _BPW)], e_idx_v)

    def _mod_body(i, carry):
        sl = pl.ds(i * _L, _L)
        s_idx_v[sl] = lax.rem(s_idx_v[sl], VOCAB)
        e_idx_v[sl] = lax.rem(e_idx_v[sl], VOCAB)
        return carry

    lax.fori_loop(0, _BPW // _L, _mod_body, 0)

    sems = (sem0, sem1)
    def _gather(c, buf):
        if c < _NCH:
            table, idx = s_table, s_idx_v.at[pl.ds(c * _CH, _CH)]
        else:
            table, idx = e_table, e_idx_v.at[pl.ds((c - _NCH) * _CH, _CH)]
        return pltpu.async_copy(table.at[idx], rows_v.at[buf], sems[buf])

    cps = [_gather(0, 0), _gather(1, 1)]
    for c in range(2 * _NCH):
        buf = c % 2
        cps[buf].wait()
        if c + 2 < 2 * _NCH:
            nxt = c + 2
        else:
            nxt = None
        if c < _NCH:
            out, row0 = s_out, base + c * _CH
        else:
            out, row0 = e_out, base + (c - _NCH) * _CH
        pltpu.sync_copy(rows_v.at[buf], out.at[pl.ds(row0, _CH)])
        if nxt is not None:
            cps[buf] = _gather(nxt, buf)


def _l2norm(x):
    sq = jnp.sum(x * x, axis=-1, keepdims=True)
    return x * lax.rsqrt(jnp.maximum(sq, 1e-12))


def _tower(x, W1, b1, W2, b2, W3, b3):
    h = jnp.maximum(jnp.dot(x, W1, preferred_element_type=jnp.float32) + b1, 0.0)
    h = jnp.maximum(jnp.dot(h, W2, preferred_element_type=jnp.float32) + b2, 0.0)
    return jnp.maximum(jnp.dot(h, W3, preferred_element_type=jnp.float32) + b3, 0.0)


def _tc_body(sf_ref, ef_ref, sb_ref, eb_ref,
             sW1, sb1, sW2, sb2, sW3, sb3,
             eW1, eb1, eW2, eb2, eW3, eb3,
             rW, rb, lW, lb, kW, kb,
             rank_ref, like_ref, risk_ref):
    s_feat = _tower(sf_ref[...], sW1[...], sb1[...], sW2[...], sb2[...],
                    sW3[...], sb3[...])
    e_feat = _tower(ef_ref[...], eW1[...], eb1[...], eW2[...], eb2[...],
                    eW3[...], eb3[...])
    s_emb = _l2norm(sb_ref[:, :D] + s_feat)
    e_emb = _l2norm(eb_ref[:, :D] + e_feat)
    combined = s_emb * e_emb

    def _head(W, b, out_ref):
        z = jnp.dot(combined, W[...], preferred_element_type=jnp.float32) + b[...]
        out_ref[...] = (1.0 / (1.0 + jnp.exp(-z)))[:, 0]

    _head(rW, rb, rank_ref)
    _head(lW, lb, like_ref)
    _head(kW, kb, risk_ref)


_R = 2048


def _tc_dense(sf, ef, sb, eb, weights):
    def _full(a):
        return pl.BlockSpec(a.shape, lambda i: (0,) * a.ndim)

    row_spec = lambda a: pl.BlockSpec((_R, a.shape[1]), lambda i: (i, 0))
    base_spec = pl.BlockSpec((_R, _DP), lambda i: (i, 0))
    in_specs = [row_spec(sf), row_spec(ef), base_spec, base_spec] + [
        _full(w) for w in weights
    ]
    out_spec = pl.BlockSpec((_R,), lambda i: (i,))
    out_shape = jax.ShapeDtypeStruct((B,), jnp.float32)
    return pl.pallas_call(
        _tc_body,
        grid=(B // _R,),
        in_specs=in_specs,
        out_specs=[out_spec] * 3,
        out_shape=[out_shape] * 3,
    )(sf, ef, sb, eb, *weights)


def kernel(student_id, engagement_id, student_features, engagement_features,
           student_table, engagement_table,
           s_W1, s_b1, s_W2, s_b2, s_W3, s_b3,
           e_W1, e_b1, e_W2, e_b2, e_W3, e_b3,
           rank_W, rank_b, like_W, like_b, risk_W, risk_b):
    s_table_p = jnp.pad(student_table, ((0, 0), (0, _DP - D)))
    e_table_p = jnp.pad(engagement_table, ((0, 0), (0, _DP - D)))
    s_base, e_base = _sc_gather(s_table_p, e_table_p,
                                student_id, engagement_id)
    weights = (s_W1, s_b1, s_W2, s_b2, s_W3, s_b3,
               e_W1, e_b1, e_W2, e_b2, e_W3, e_b3,
               rank_W, rank_b, like_W, like_b, risk_W, risk_b)
    rank, like, risk = _tc_dense(student_features, engagement_features,
                                 s_base, e_base, weights)
    return (rank[:, None], like[:, None], risk[:, None])
